# baseline (speedup 1.0000x reference)
import functools

import jax
import jax.numpy as jnp
from jax.experimental import pallas as pl
from jax.experimental.pallas import tpu as pltpu

EMB = 1024
NUM_EXPERTS = 8
TOP_K = 2
HID = 2048
B, S = 2, 2048
T = B * S
EPAD = 128
LOAD_COEFF = 0.1
Z_ROUTER_COEFF = 0.001

_SQRT_2_OVER_PI = 0.7978845608028654


def _gelu_tanh(x):
    return 0.5 * x * (1.0 + jnp.tanh(_SQRT_2_OVER_PI * (x + 0.044715 * x * x * x)))


def _router_body(x_ref, gw_ref, gb_ref, wgt_ref, loss_ref):
    x = x_ref[...]
    logits = jax.lax.dot_general(
        x, gw_ref[...], (((1,), (0,)), ((), ())),
        preferred_element_type=jnp.float32) + gb_ref[...]
    lane = jax.lax.broadcasted_iota(jnp.int32, (T, EPAD), 1)
    valid = lane < NUM_EXPERTS
    lm = jnp.where(valid, logits, -1e30)
    m = jnp.max(lm, axis=1, keepdims=True)
    ex = jnp.where(valid, jnp.exp(lm - m), 0.0)
    denom = jnp.sum(ex, axis=1, keepdims=True)
    probs = ex / denom
    lse = m + jnp.log(denom)

    p1 = jnp.max(probs, axis=1, keepdims=True)
    i1 = jnp.min(jnp.where(probs == p1, lane, EPAD), axis=1, keepdims=True)
    mask1 = lane == i1
    p2 = jnp.max(jnp.where(mask1, -1.0, probs), axis=1, keepdims=True)
    i2 = jnp.min(jnp.where((probs == p2) & (~mask1), lane, EPAD),
                 axis=1, keepdims=True)
    ssum = p1 + p2
    w1p = p1 / ssum
    w2p = p2 / ssum
    onehot = (mask1).astype(jnp.float32) + (lane == i2).astype(jnp.float32)
    wgt_ref[...] = jnp.where(mask1, w1p, 0.0) + jnp.where(lane == i2, w2p, 0.0)

    z_loss = jnp.sum(lse * lse) * (1.0 / T)
    counts = jnp.sum(onehot, axis=0, keepdims=True)
    p_mean = jnp.sum(probs, axis=0, keepdims=True) * (1.0 / T)
    f_i = counts * (1.0 / (TOP_K * T))
    load_loss = NUM_EXPERTS * jnp.sum(f_i * p_mean)
    loss_ref[0, 0] = Z_ROUTER_COEFF * z_loss + LOAD_COEFF * load_loss


def _router(x2, gw_pad, gb_pad):
    return pl.pallas_call(
        _router_body,
        out_shape=(
            jax.ShapeDtypeStruct((T, EPAD), jnp.float32),
            jax.ShapeDtypeStruct((1, 1), jnp.float32),
        ),
        in_specs=[
            pl.BlockSpec((T, EMB), lambda: (0, 0)),
            pl.BlockSpec((EMB, EPAD), lambda: (0, 0)),
            pl.BlockSpec((1, EPAD), lambda: (0, 0)),
        ],
        out_specs=(
            pl.BlockSpec((T, EPAD), lambda: (0, 0)),
            pl.BlockSpec(memory_space=pltpu.SMEM),
        ),
    )(x2, gw_pad, gb_pad)


BT = 512
NT = T // BT


def _ffn_body(x_ref, w1_ref, b1_ref, w2_ref, b2_ref, wgt_ref, out_ref):
    e = pl.program_id(0)
    t = pl.program_id(1)
    xb = x_ref[...]
    h = jax.lax.dot_general(
        xb, w1_ref[0], (((1,), (0,)), ((), ())),
        preferred_element_type=jnp.float32) + b1_ref[0]
    h = _gelu_tanh(h)
    y = jax.lax.dot_general(
        h, w2_ref[0], (((1,), (0,)), ((), ())),
        preferred_element_type=jnp.float32) + b2_ref[0]
    lane = jax.lax.broadcasted_iota(jnp.int32, (BT, EPAD), 1)
    w = jnp.sum(jnp.where(lane == e, wgt_ref[...], 0.0), axis=1, keepdims=True)
    contrib = y * w
    row0 = t * BT

    @pl.when(e == 0)
    def _():
        out_ref[pl.ds(row0, BT), :] = contrib

    @pl.when(e > 0)
    def _():
        out_ref[pl.ds(row0, BT), :] += contrib


def _ffn(x2, w1, b1, w2, b2, wgt):
    return pl.pallas_call(
        _ffn_body,
        grid=(NUM_EXPERTS, NT),
        in_specs=[
            pl.BlockSpec((BT, EMB), lambda e, t: (t, 0)),
            pl.BlockSpec((1, EMB, HID), lambda e, t: (e, 0, 0)),
            pl.BlockSpec((1, 1, HID), lambda e, t: (e, 0, 0)),
            pl.BlockSpec((1, HID, EMB), lambda e, t: (e, 0, 0)),
            pl.BlockSpec((1, 1, EMB), lambda e, t: (e, 0, 0)),
            pl.BlockSpec((BT, EPAD), lambda e, t: (t, 0)),
        ],
        out_specs=pl.BlockSpec((T, EMB), lambda e, t: (0, 0)),
        out_shape=jax.ShapeDtypeStruct((T, EMB), jnp.float32),
        compiler_params=pltpu.CompilerParams(
            dimension_semantics=("arbitrary", "arbitrary")),
    )(x2, w1, b1, w2, b2, wgt)


@jax.jit
def kernel(x, gate_w, gate_b, w1, b1, w2, b2):
    x2 = x.reshape(T, EMB)
    gw_pad = jnp.pad(gate_w, ((0, 0), (0, EPAD - NUM_EXPERTS)))
    gb_pad = jnp.pad(gate_b, (0, EPAD - NUM_EXPERTS)).reshape(1, EPAD)
    wgt, loss = _router(x2, gw_pad, gb_pad)
    out = _ffn(x2.astype(jnp.bfloat16), w1.astype(jnp.bfloat16),
               b1.reshape(NUM_EXPERTS, 1, HID), w2.astype(jnp.bfloat16),
               b2.reshape(NUM_EXPERTS, 1, EMB), wgt)
    return out.reshape(B, S, EMB), loss[0, 0]


# speedup vs baseline: 1.0731x; 1.0731x over previous
import functools

import jax
import jax.numpy as jnp
from jax import lax
from jax.experimental import pallas as pl
from jax.experimental.pallas import tpu as pltpu
from jax.experimental.pallas import tpu_sc as plsc

EMB = 1024
NUM_EXPERTS = 8
TOP_K = 2
HID = 2048
B, S = 2, 2048
T = B * S
A = T * TOP_K
EPAD = 128
LOAD_COEFF = 0.1
Z_ROUTER_COEFF = 0.001

BT = 128
NTILES = (A + NUM_EXPERTS * (BT - 1) + BT - 1) // BT
NPAD = NTILES * BT

NC, NS = 2, 16
NW = NC * NS

_SQRT_2_OVER_PI = 0.7978845608028654


def _gelu_tanh(x):
    return 0.5 * x * (1.0 + jnp.tanh(_SQRT_2_OVER_PI * (x + 0.044715 * x * x * x)))


def _router_body(x_ref, gw_ref, gb_ref, ti_ref, wp_ref, loss_ref):
    x = x_ref[...]
    logits = lax.dot_general(
        x, gw_ref[...], (((1,), (0,)), ((), ())),
        preferred_element_type=jnp.float32) + gb_ref[...]
    lane = lax.broadcasted_iota(jnp.int32, (T, EPAD), 1)
    valid = lane < NUM_EXPERTS
    lm = jnp.where(valid, logits, -1e30)
    m = jnp.max(lm, axis=1, keepdims=True)
    ex = jnp.where(valid, jnp.exp(lm - m), 0.0)
    denom = jnp.sum(ex, axis=1, keepdims=True)
    probs = ex / denom
    lse = m + jnp.log(denom)

    p1 = jnp.max(probs, axis=1, keepdims=True)
    i1 = jnp.min(jnp.where(probs == p1, lane, EPAD), axis=1, keepdims=True)
    mask1 = lane == i1
    p2 = jnp.max(jnp.where(mask1, -1.0, probs), axis=1, keepdims=True)
    i2 = jnp.min(jnp.where((probs == p2) & (~mask1), lane, EPAD),
                 axis=1, keepdims=True)
    ssum = p1 + p2
    ti_ref[...] = jnp.concatenate([i1, i2], axis=1)
    wp_ref[...] = jnp.concatenate([p1 / ssum, p2 / ssum], axis=1)

    onehot = mask1.astype(jnp.float32) + (lane == i2).astype(jnp.float32)
    z_loss = jnp.sum(lse * lse) * (1.0 / T)
    counts = jnp.sum(onehot, axis=0, keepdims=True)
    p_mean = jnp.sum(probs, axis=0, keepdims=True) * (1.0 / T)
    f_i = counts * (1.0 / (TOP_K * T))
    load_loss = NUM_EXPERTS * jnp.sum(f_i * p_mean)
    loss_ref[0, 0] = Z_ROUTER_COEFF * z_loss + LOAD_COEFF * load_loss


def _router(x2, gw_pad, gb_pad):
    return pl.pallas_call(
        _router_body,
        out_shape=(
            jax.ShapeDtypeStruct((T, TOP_K), jnp.int32),
            jax.ShapeDtypeStruct((T, TOP_K), jnp.float32),
            jax.ShapeDtypeStruct((1, 1), jnp.float32),
        ),
        in_specs=[
            pl.BlockSpec((T, EMB), lambda: (0, 0)),
            pl.BlockSpec((EMB, EPAD), lambda: (0, 0)),
            pl.BlockSpec((1, EPAD), lambda: (0, 0)),
        ],
        out_specs=(
            pl.BlockSpec((T, TOP_K), lambda: (0, 0)),
            pl.BlockSpec((T, TOP_K), lambda: (0, 0)),
            pl.BlockSpec(memory_space=pltpu.SMEM),
        ),
    )(x2, gw_pad, gb_pad)


def _dispatch_metadata(top_idx, wpair):
    e_flat = top_idx.reshape(A)
    oneh = (e_flat[:, None] == jnp.arange(NUM_EXPERTS)[None, :]).astype(jnp.int32)
    g = jnp.sum(oneh, axis=0)
    pg = ((g + BT - 1) // BT) * BT
    ends = jnp.cumsum(pg)
    off = ends - pg
    rank = jnp.cumsum(oneh, axis=0) - oneh
    dest = jnp.sum(oneh * (rank + off[None, :]), axis=1).astype(jnp.int32)
    tok = (jnp.arange(A, dtype=jnp.int32) // TOP_K)
    idx_pad = jnp.zeros((NPAD,), jnp.int32).at[dest].set(tok)
    w_pad = jnp.zeros((NPAD,), jnp.float32).at[dest].set(wpair.reshape(A))
    pos = dest.reshape(T, TOP_K)
    tile_starts = jnp.arange(NTILES, dtype=jnp.int32) * BT
    tile_e = jnp.minimum(
        jnp.searchsorted(ends, tile_starts, side="right"),
        NUM_EXPERTS - 1).astype(jnp.int32)
    return idx_pad, w_pad[:, None], pos, tile_e


_GCH = 48
_RPW = NPAD // NW


def _gather_body(x_hbm, idx_hbm, out_hbm, idx_v, rows_v, sem):
    wid = lax.axis_index("s") * NC + lax.axis_index("c")
    base = wid * _RPW
    for c in range(_RPW // _GCH):
        b = base + c * _GCH
        pltpu.sync_copy(idx_hbm.at[pl.ds(b, _GCH)], idx_v)
        pltpu.async_copy(x_hbm.at[idx_v], rows_v, sem).wait()
        pltpu.sync_copy(rows_v, out_hbm.at[pl.ds(b, _GCH)])


def _sc_gather(x2, idx_pad):
    mesh = plsc.VectorSubcoreMesh(core_axis_name="c", subcore_axis_name="s")
    fn = functools.partial(
        pl.kernel,
        mesh=mesh,
        out_type=jax.ShapeDtypeStruct((NPAD, EMB), jnp.float32),
        scratch_types=[
            pltpu.VMEM((_GCH,), jnp.int32),
            pltpu.VMEM((_GCH, EMB), jnp.float32),
            pltpu.SemaphoreType.DMA,
        ],
    )(_gather_body)
    return fn(x2, idx_pad)


def _ffn_body(te_ref, x_ref, w1_ref, b1_ref, w2_ref, b2_ref, wp_ref, y_ref):
    h = lax.dot_general(
        x_ref[...], w1_ref[0], (((1,), (0,)), ((), ())),
        preferred_element_type=jnp.float32) + b1_ref[0]
    h = _gelu_tanh(h)
    y = lax.dot_general(
        h, w2_ref[0], (((1,), (0,)), ((), ())),
        preferred_element_type=jnp.float32) + b2_ref[0]
    y_ref[...] = y * wp_ref[...]


def _ffn(x_pad, w1, b1, w2, b2, w_pad, tile_e):
    grid_spec = pltpu.PrefetchScalarGridSpec(
        num_scalar_prefetch=1,
        grid=(NTILES,),
        in_specs=[
            pl.BlockSpec((BT, EMB), lambda i, te: (i, 0)),
            pl.BlockSpec((1, EMB, HID), lambda i, te: (te[i], 0, 0)),
            pl.BlockSpec((1, 1, HID), lambda i, te: (te[i], 0, 0)),
            pl.BlockSpec((1, HID, EMB), lambda i, te: (te[i], 0, 0)),
            pl.BlockSpec((1, 1, EMB), lambda i, te: (te[i], 0, 0)),
            pl.BlockSpec((BT, 1), lambda i, te: (i, 0)),
        ],
        out_specs=pl.BlockSpec((BT, EMB), lambda i, te: (i, 0)),
    )
    return pl.pallas_call(
        _ffn_body,
        grid_spec=grid_spec,
        out_shape=jax.ShapeDtypeStruct((NPAD, EMB), jnp.float32),
        compiler_params=pltpu.CompilerParams(
            dimension_semantics=("arbitrary",)),
    )(tile_e, x_pad, w1, b1, w2, b2, w_pad)


_CCH = 32
_TPW = T // NW


def _combine_body(y_hbm, p0_hbm, p1_hbm, out_hbm,
                  i0_v, i1_v, a_v, b_v, sem0, sem1):
    wid = lax.axis_index("s") * NC + lax.axis_index("c")
    base = wid * _TPW
    for c in range(_TPW // _CCH):
        b = base + c * _CCH
        pltpu.sync_copy(p0_hbm.at[pl.ds(b, _CCH)], i0_v)
        pltpu.sync_copy(p1_hbm.at[pl.ds(b, _CCH)], i1_v)
        cp0 = pltpu.async_copy(y_hbm.at[i0_v], a_v, sem0)
        cp1 = pltpu.async_copy(y_hbm.at[i1_v], b_v, sem1)
        cp0.wait()
        cp1.wait()
        for r in range(_CCH):
            def add_row(i, _, r=r):
                sl = pl.ds(i * 16, 16)
                a_v[r, sl] = a_v[r, sl] + b_v[r, sl]
                return 0
            lax.fori_loop(0, EMB // 16, add_row, 0)
        pltpu.sync_copy(a_v, out_hbm.at[pl.ds(b, _CCH)])


def _sc_combine(y_pad, pos):
    mesh = plsc.VectorSubcoreMesh(core_axis_name="c", subcore_axis_name="s")
    fn = functools.partial(
        pl.kernel,
        mesh=mesh,
        out_type=jax.ShapeDtypeStruct((T, EMB), jnp.float32),
        scratch_types=[
            pltpu.VMEM((_CCH,), jnp.int32),
            pltpu.VMEM((_CCH,), jnp.int32),
            pltpu.VMEM((_CCH, EMB), jnp.float32),
            pltpu.VMEM((_CCH, EMB), jnp.float32),
            pltpu.SemaphoreType.DMA,
            pltpu.SemaphoreType.DMA,
        ],
    )(_combine_body)
    p0 = pos[:, 0]
    p1 = pos[:, 1]
    return fn(y_pad, p0, p1)


@jax.jit
def kernel(x, gate_w, gate_b, w1, b1, w2, b2):
    x2 = x.reshape(T, EMB)
    gw_pad = jnp.pad(gate_w, ((0, 0), (0, EPAD - NUM_EXPERTS)))
    gb_pad = jnp.pad(gate_b, (0, EPAD - NUM_EXPERTS)).reshape(1, EPAD)
    top_idx, wpair, loss = _router(x2, gw_pad, gb_pad)
    idx_pad, w_pad, pos, tile_e = _dispatch_metadata(top_idx, wpair)
    x_pad = _sc_gather(x2, idx_pad)
    y_pad = _ffn(x_pad, w1, b1.reshape(NUM_EXPERTS, 1, HID),
                 w2, b2.reshape(NUM_EXPERTS, 1, EMB), w_pad, tile_e)
    out = _sc_combine(y_pad, pos)
    return out.reshape(B, S, EMB), loss[0, 0]


# speedup vs baseline: 1.1229x; 1.0464x over previous
import functools

import jax
import jax.numpy as jnp
from jax import lax
from jax.experimental import pallas as pl
from jax.experimental.pallas import tpu as pltpu
from jax.experimental.pallas import tpu_sc as plsc

EMB = 1024
NUM_EXPERTS = 8
TOP_K = 2
HID = 2048
B, S = 2, 2048
T = B * S
A = T * TOP_K
EPAD = 128
LOAD_COEFF = 0.1
Z_ROUTER_COEFF = 0.001

BT = 128
NTILES = (A + NUM_EXPERTS * (BT - 1) + BT - 1) // BT
NPAD = NTILES * BT

NC, NS = 2, 16
NW = NC * NS

_SQRT_2_OVER_PI = 0.7978845608028654


def _gelu_tanh(x):
    return 0.5 * x * (1.0 + jnp.tanh(_SQRT_2_OVER_PI * (x + 0.044715 * x * x * x)))


def _router_body(x_ref, gw_ref, gb_ref, ti_ref, wp_ref, loss_ref):
    x = x_ref[...]
    logits = lax.dot_general(
        x, gw_ref[...], (((1,), (0,)), ((), ())),
        preferred_element_type=jnp.float32) + gb_ref[...]
    lane = lax.broadcasted_iota(jnp.int32, (T, EPAD), 1)
    valid = lane < NUM_EXPERTS
    lm = jnp.where(valid, logits, -1e30)
    m = jnp.max(lm, axis=1, keepdims=True)
    ex = jnp.where(valid, jnp.exp(lm - m), 0.0)
    denom = jnp.sum(ex, axis=1, keepdims=True)
    probs = ex / denom
    lse = m + jnp.log(denom)

    p1 = jnp.max(probs, axis=1, keepdims=True)
    i1 = jnp.min(jnp.where(probs == p1, lane, EPAD), axis=1, keepdims=True)
    mask1 = lane == i1
    p2 = jnp.max(jnp.where(mask1, -1.0, probs), axis=1, keepdims=True)
    i2 = jnp.min(jnp.where((probs == p2) & (~mask1), lane, EPAD),
                 axis=1, keepdims=True)
    ssum = p1 + p2
    ti_ref[...] = jnp.concatenate([i1, i2], axis=1)
    wp_ref[...] = jnp.concatenate([p1 / ssum, p2 / ssum], axis=1)

    onehot = mask1.astype(jnp.float32) + (lane == i2).astype(jnp.float32)
    z_loss = jnp.sum(lse * lse) * (1.0 / T)
    counts = jnp.sum(onehot, axis=0, keepdims=True)
    p_mean = jnp.sum(probs, axis=0, keepdims=True) * (1.0 / T)
    f_i = counts * (1.0 / (TOP_K * T))
    load_loss = NUM_EXPERTS * jnp.sum(f_i * p_mean)
    loss_ref[0, 0] = Z_ROUTER_COEFF * z_loss + LOAD_COEFF * load_loss


def _router(x2, gw_pad, gb_pad):
    return pl.pallas_call(
        _router_body,
        out_shape=(
            jax.ShapeDtypeStruct((T, TOP_K), jnp.int32),
            jax.ShapeDtypeStruct((T, TOP_K), jnp.float32),
            jax.ShapeDtypeStruct((1, 1), jnp.float32),
        ),
        in_specs=[
            pl.BlockSpec((T, EMB), lambda: (0, 0)),
            pl.BlockSpec((EMB, EPAD), lambda: (0, 0)),
            pl.BlockSpec((1, EPAD), lambda: (0, 0)),
        ],
        out_specs=(
            pl.BlockSpec((T, TOP_K), lambda: (0, 0)),
            pl.BlockSpec((T, TOP_K), lambda: (0, 0)),
            pl.BlockSpec(memory_space=pltpu.SMEM),
        ),
    )(x2, gw_pad, gb_pad)


def _dispatch_metadata(top_idx, wpair):
    e_flat = top_idx.reshape(A)
    oneh = (e_flat[:, None] == jnp.arange(NUM_EXPERTS)[None, :]).astype(jnp.int32)
    g = jnp.sum(oneh, axis=0)
    pg = ((g + BT - 1) // BT) * BT
    ends = jnp.cumsum(pg)
    off = ends - pg
    rank = jnp.cumsum(oneh, axis=0) - oneh
    dest = jnp.sum(oneh * (rank + off[None, :]), axis=1).astype(jnp.int32)
    tok = (jnp.arange(A, dtype=jnp.int32) // TOP_K)
    idx_pad = jnp.zeros((NPAD,), jnp.int32).at[dest].set(tok)
    w_pad = jnp.zeros((NPAD,), jnp.float32).at[dest].set(wpair.reshape(A))
    pos = dest.reshape(T, TOP_K)
    tile_starts = jnp.arange(NTILES, dtype=jnp.int32) * BT
    tile_e = jnp.minimum(
        jnp.searchsorted(ends, tile_starts, side="right"),
        NUM_EXPERTS - 1).astype(jnp.int32)
    return idx_pad, w_pad[:, None], pos, tile_e


_GCH = 48
_RPW = NPAD // NW
_GN = _RPW // _GCH


def _gather_body(x_hbm, idx_hbm, out_hbm, idx_v, r0, r1,
                 gs0, gs1, ws0, ws1):
    wid = lax.axis_index("s") * NC + lax.axis_index("c")
    base = wid * _RPW
    pltpu.sync_copy(idx_hbm.at[pl.ds(base, _RPW)], idx_v)
    bufs = (r0, r1)
    gsems = (gs0, gs1)
    wsems = (ws0, ws1)
    gcp = [None] * _GN
    wcp = [None] * _GN
    gcp[0] = pltpu.async_copy(x_hbm.at[idx_v.at[pl.ds(0, _GCH)]], r0, gs0)
    for c in range(_GN):
        nb = (c + 1) % 2
        if c + 1 < _GN:
            if c >= 1:
                wcp[c - 1].wait()
            gcp[c + 1] = pltpu.async_copy(
                x_hbm.at[idx_v.at[pl.ds((c + 1) * _GCH, _GCH)]],
                bufs[nb], gsems[nb])
        gcp[c].wait()
        wcp[c] = pltpu.async_copy(
            bufs[c % 2], out_hbm.at[pl.ds(base + c * _GCH, _GCH)],
            wsems[c % 2])
    wcp[_GN - 2].wait()
    wcp[_GN - 1].wait()


def _sc_gather(x2, idx_pad):
    mesh = plsc.VectorSubcoreMesh(core_axis_name="c", subcore_axis_name="s")
    fn = functools.partial(
        pl.kernel,
        mesh=mesh,
        out_type=jax.ShapeDtypeStruct((NPAD, EMB), jnp.float32),
        scratch_types=[
            pltpu.VMEM((_RPW,), jnp.int32),
            pltpu.VMEM((_GCH, EMB), jnp.float32),
            pltpu.VMEM((_GCH, EMB), jnp.float32),
            pltpu.SemaphoreType.DMA,
            pltpu.SemaphoreType.DMA,
            pltpu.SemaphoreType.DMA,
            pltpu.SemaphoreType.DMA,
        ],
    )(_gather_body)
    return fn(x2, idx_pad)


def _ffn_body(te_ref, x_ref, w1_ref, b1_ref, w2_ref, b2_ref, wp_ref, y_ref):
    h = lax.dot_general(
        x_ref[...], w1_ref[0], (((1,), (0,)), ((), ())),
        preferred_element_type=jnp.float32) + b1_ref[0]
    h = _gelu_tanh(h)
    y = lax.dot_general(
        h, w2_ref[0], (((1,), (0,)), ((), ())),
        preferred_element_type=jnp.float32) + b2_ref[0]
    y_ref[...] = y * wp_ref[...]


def _ffn(x_pad, w1, b1, w2, b2, w_pad, tile_e):
    grid_spec = pltpu.PrefetchScalarGridSpec(
        num_scalar_prefetch=1,
        grid=(NTILES,),
        in_specs=[
            pl.BlockSpec((BT, EMB), lambda i, te: (i, 0)),
            pl.BlockSpec((1, EMB, HID), lambda i, te: (te[i], 0, 0)),
            pl.BlockSpec((1, 1, HID), lambda i, te: (te[i], 0, 0)),
            pl.BlockSpec((1, HID, EMB), lambda i, te: (te[i], 0, 0)),
            pl.BlockSpec((1, 1, EMB), lambda i, te: (te[i], 0, 0)),
            pl.BlockSpec((BT, 1), lambda i, te: (i, 0)),
        ],
        out_specs=pl.BlockSpec((BT, EMB), lambda i, te: (i, 0)),
    )
    return pl.pallas_call(
        _ffn_body,
        grid_spec=grid_spec,
        out_shape=jax.ShapeDtypeStruct((NPAD, EMB), jnp.float32),
        compiler_params=pltpu.CompilerParams(
            dimension_semantics=("arbitrary",)),
    )(tile_e, x_pad, w1, b1, w2, b2, w_pad)


_CCH = 16
_TPW = T // NW
_CN = _TPW // _CCH


def _combine_body(y_hbm, p0_hbm, p1_hbm, out_hbm, i0_v, i1_v,
                  a0, b0, a1, b1, sa0, sb0, sa1, sb1, ws0, ws1):
    wid = lax.axis_index("s") * NC + lax.axis_index("c")
    base = wid * _TPW
    pltpu.sync_copy(p0_hbm.at[pl.ds(base, _TPW)], i0_v)
    pltpu.sync_copy(p1_hbm.at[pl.ds(base, _TPW)], i1_v)
    abufs, bbufs = (a0, a1), (b0, b1)
    asems, bsems, wsems = (sa0, sa1), (sb0, sb1), (ws0, ws1)

    def issue(c):
        k = c % 2
        sl = pl.ds(c * _CCH, _CCH)
        return (
            pltpu.async_copy(y_hbm.at[i0_v.at[sl]], abufs[k], asems[k]),
            pltpu.async_copy(y_hbm.at[i1_v.at[sl]], bbufs[k], bsems[k]),
        )

    g = [None] * _CN
    w = [None] * _CN
    g[0] = issue(0)
    for c in range(_CN):
        k = c % 2
        if c + 1 < _CN:
            if c >= 1:
                w[c - 1].wait()
            g[c + 1] = issue(c + 1)
        g[c][0].wait()
        g[c][1].wait()
        av, bv = abufs[k], bbufs[k]
        for r in range(_CCH):
            def add_row(i, _, r=r, av=av, bv=bv):
                for u in range(4):
                    sl = pl.ds((i * 4 + u) * 16, 16)
                    av[r, sl] = av[r, sl] + bv[r, sl]
                return 0
            lax.fori_loop(0, EMB // 64, add_row, 0)
        w[c] = pltpu.async_copy(
            av, out_hbm.at[pl.ds(base + c * _CCH, _CCH)], wsems[k])
    w[_CN - 2].wait()
    w[_CN - 1].wait()


def _sc_combine(y_pad, pos):
    mesh = plsc.VectorSubcoreMesh(core_axis_name="c", subcore_axis_name="s")
    fn = functools.partial(
        pl.kernel,
        mesh=mesh,
        out_type=jax.ShapeDtypeStruct((T, EMB), jnp.float32),
        scratch_types=[
            pltpu.VMEM((_TPW,), jnp.int32),
            pltpu.VMEM((_TPW,), jnp.int32),
            pltpu.VMEM((_CCH, EMB), jnp.float32),
            pltpu.VMEM((_CCH, EMB), jnp.float32),
            pltpu.VMEM((_CCH, EMB), jnp.float32),
            pltpu.VMEM((_CCH, EMB), jnp.float32),
            pltpu.SemaphoreType.DMA,
            pltpu.SemaphoreType.DMA,
            pltpu.SemaphoreType.DMA,
            pltpu.SemaphoreType.DMA,
            pltpu.SemaphoreType.DMA,
            pltpu.SemaphoreType.DMA,
        ],
    )(_combine_body)
    p0 = pos[:, 0]
    p1 = pos[:, 1]
    return fn(y_pad, p0, p1)


@jax.jit
def kernel(x, gate_w, gate_b, w1, b1, w2, b2):
    x2 = x.reshape(T, EMB)
    gw_pad = jnp.pad(gate_w, ((0, 0), (0, EPAD - NUM_EXPERTS)))
    gb_pad = jnp.pad(gate_b, (0, EPAD - NUM_EXPERTS)).reshape(1, EPAD)
    top_idx, wpair, loss = _router(x2, gw_pad, gb_pad)
    idx_pad, w_pad, pos, tile_e = _dispatch_metadata(top_idx, wpair)
    x_pad = _sc_gather(x2, idx_pad)
    y_pad = _ffn(x_pad, w1, b1.reshape(NUM_EXPERTS, 1, HID),
                 w2, b2.reshape(NUM_EXPERTS, 1, EMB), w_pad, tile_e)
    out = _sc_combine(y_pad, pos)
    return out.reshape(B, S, EMB), loss[0, 0]
